# Initial kernel scaffold; baseline (speedup 1.0000x reference)
import jax
import jax.numpy as jnp
from jax.experimental import pallas as pl


def kernel(nf, ef, edge_index, W_node_enc, b_node_enc, W_edge_enc, b_edge_enc, We, be, Wn, bn):
    raise NotImplementedError("write your pallas kernel here")


# speedup vs baseline: 2.9092x; 2.9092x over previous
import functools

import jax
import jax.numpy as jnp
from jax import lax
from jax.experimental import pallas as pl
from jax.experimental.pallas import tpu as pltpu
from jax.experimental.pallas import tpu_sc as plsc

N = 10000
E = 320000
NODE_DIM = 128
EDGE_DIM = 16
D = 128
N_LAYER = 3

NC = 2
NS = 16
NW = NC * NS
EPW = E // NW
C = 80
NCHUNK = EPW // C
SN = 624
SREM = N - NS * SN

BN = 2000
BE = 2000

_mesh = plsc.VectorSubcoreMesh(
    core_axis_name="c", subcore_axis_name="s", num_cores=NC, num_subcores=NS)


def _dot(a, b):
    return jnp.dot(a, b, preferred_element_type=jnp.float32)


def _edge_enc_body(x_ref, w_ref, b_ref, o_ref):
    o_ref[...] = _dot(x_ref[...], w_ref[...]) + b_ref[...]


_edge_enc = pl.pallas_call(
    _edge_enc_body,
    grid=(E // BE,),
    in_specs=[
        pl.BlockSpec((BE, EDGE_DIM), lambda i: (i, 0)),
        pl.BlockSpec((EDGE_DIM, D), lambda i: (0, 0)),
        pl.BlockSpec((1, D), lambda i: (0, 0)),
    ],
    out_specs=pl.BlockSpec((BE, D), lambda i: (i, 0)),
    out_shape=jax.ShapeDtypeStruct((E, D), jnp.float32),
)


def _node_enc_body(x_ref, w_ref, b_ref, ws_ref, bs_ref, wd_ref,
                   u_ref, p_ref, q_ref):
    u = _dot(x_ref[...], w_ref[...]) + b_ref[...]
    u_ref[...] = u
    p_ref[...] = _dot(u, ws_ref[...]) + bs_ref[...]
    q_ref[...] = _dot(u, wd_ref[...])


_node_enc = pl.pallas_call(
    _node_enc_body,
    grid=(N // BN,),
    in_specs=[
        pl.BlockSpec((BN, NODE_DIM), lambda i: (i, 0)),
        pl.BlockSpec((NODE_DIM, D), lambda i: (0, 0)),
        pl.BlockSpec((1, D), lambda i: (0, 0)),
        pl.BlockSpec((D, D), lambda i: (0, 0)),
        pl.BlockSpec((1, D), lambda i: (0, 0)),
        pl.BlockSpec((D, D), lambda i: (0, 0)),
    ],
    out_specs=[pl.BlockSpec((BN, D), lambda i: (i, 0))] * 3,
    out_shape=[jax.ShapeDtypeStruct((N, D), jnp.float32)] * 3,
)


def _edge_upd_body(u_ref, gs_ref, gd_ref, w_ref, o_ref):
    u = u_ref[...]
    pre = _dot(u, w_ref[...]) + gs_ref[...] + gd_ref[...]
    o_ref[...] = u + jnp.maximum(pre, 0.0)


_edge_upd = pl.pallas_call(
    _edge_upd_body,
    grid=(E // BE,),
    in_specs=[
        pl.BlockSpec((BE, D), lambda i: (i, 0)),
        pl.BlockSpec((BE, D), lambda i: (i, 0)),
        pl.BlockSpec((BE, D), lambda i: (i, 0)),
        pl.BlockSpec((D, D), lambda i: (0, 0)),
    ],
    out_specs=pl.BlockSpec((BE, D), lambda i: (i, 0)),
    out_shape=jax.ShapeDtypeStruct((E, D), jnp.float32),
)


def _node_upd_proj_body(u_ref, a0_ref, a1_ref, w1_ref, w2_ref, b_ref,
                        ws_ref, bs_ref, wd_ref, uo_ref, p_ref, q_ref):
    u = u_ref[...]
    agg = a0_ref[...] + a1_ref[...]
    h = _dot(u, w1_ref[...]) + _dot(agg, w2_ref[...]) + b_ref[...]
    un = u + jnp.maximum(h, 0.0)
    uo_ref[...] = un
    p_ref[...] = _dot(un, ws_ref[...]) + bs_ref[...]
    q_ref[...] = _dot(un, wd_ref[...])


_node_upd_proj = pl.pallas_call(
    _node_upd_proj_body,
    grid=(N // BN,),
    in_specs=[
        pl.BlockSpec((BN, D), lambda i: (i, 0)),
        pl.BlockSpec((BN, D), lambda i: (i, 0)),
        pl.BlockSpec((BN, D), lambda i: (i, 0)),
        pl.BlockSpec((D, D), lambda i: (0, 0)),
        pl.BlockSpec((D, D), lambda i: (0, 0)),
        pl.BlockSpec((1, D), lambda i: (0, 0)),
        pl.BlockSpec((D, D), lambda i: (0, 0)),
        pl.BlockSpec((1, D), lambda i: (0, 0)),
        pl.BlockSpec((D, D), lambda i: (0, 0)),
    ],
    out_specs=[pl.BlockSpec((BN, D), lambda i: (i, 0))] * 3,
    out_shape=[jax.ShapeDtypeStruct((N, D), jnp.float32)] * 3,
)


def _node_upd_body(u_ref, a0_ref, a1_ref, w1_ref, w2_ref, b_ref, uo_ref):
    u = u_ref[...]
    agg = a0_ref[...] + a1_ref[...]
    h = _dot(u, w1_ref[...]) + _dot(agg, w2_ref[...]) + b_ref[...]
    uo_ref[...] = u + jnp.maximum(h, 0.0)


_node_upd = pl.pallas_call(
    _node_upd_body,
    grid=(N // BN,),
    in_specs=[
        pl.BlockSpec((BN, D), lambda i: (i, 0)),
        pl.BlockSpec((BN, D), lambda i: (i, 0)),
        pl.BlockSpec((BN, D), lambda i: (i, 0)),
        pl.BlockSpec((D, D), lambda i: (0, 0)),
        pl.BlockSpec((D, D), lambda i: (0, 0)),
        pl.BlockSpec((1, D), lambda i: (0, 0)),
    ],
    out_specs=pl.BlockSpec((BN, D), lambda i: (i, 0)),
    out_shape=jax.ShapeDtypeStruct((N, D), jnp.float32),
)


@functools.partial(
    pl.kernel,
    out_type=[jax.ShapeDtypeStruct((E, D), jnp.float32),
              jax.ShapeDtypeStruct((E, D), jnp.float32)],
    mesh=_mesh,
    scratch_types=[
        pltpu.VMEM((NCHUNK, C), jnp.int32),
        pltpu.VMEM((NCHUNK, C), jnp.int32),
        pltpu.VMEM((C, D), jnp.float32),
        pltpu.VMEM((C, D), jnp.float32),
        pltpu.SemaphoreType.DMA,
        pltpu.SemaphoreType.DMA,
    ],
)
def _sc_gather(p_hbm, q_hbm, src_hbm, dst_hbm, gs_hbm, gd_hbm,
               sidx, didx, prow, qrow, sem_p, sem_q):
    wid = lax.axis_index("s") * NC + lax.axis_index("c")
    pltpu.sync_copy(src_hbm.at[wid], sidx)
    pltpu.sync_copy(dst_hbm.at[wid], didx)

    def body(j, carry):
        base = wid * EPW + j * C
        cp_p = pltpu.async_copy(p_hbm.at[sidx.at[j]], prow, sem_p)
        cp_q = pltpu.async_copy(q_hbm.at[didx.at[j]], qrow, sem_q)
        cp_p.wait()
        cp_q.wait()
        pltpu.sync_copy(prow, gs_hbm.at[pl.ds(base, C)])
        pltpu.sync_copy(qrow, gd_hbm.at[pl.ds(base, C)])
        return carry

    lax.fori_loop(0, NCHUNK, body, 0)


@functools.partial(
    pl.kernel,
    out_type=jax.ShapeDtypeStruct((NC, N, D), jnp.float32),
    mesh=_mesh,
    scratch_types=[
        pltpu.VMEM((NCHUNK, C), jnp.int32),
        pltpu.VMEM((C, D), jnp.float32),
        pltpu.VMEM_SHARED((N, D), jnp.float32),
    ],
)
def _sc_scatter(uef_hbm, dst_hbm, zeros_hbm, out_hbm, didx, rows, acc):
    cid = lax.axis_index("c")
    sid = lax.axis_index("s")
    wid = sid * NC + cid
    pltpu.sync_copy(zeros_hbm.at[pl.ds(sid * SN, SN)],
                    acc.at[pl.ds(sid * SN, SN)])

    @pl.when(sid == NS - 1)
    def _zero_rem():
        pltpu.sync_copy(zeros_hbm.at[pl.ds(NS * SN, SREM)],
                        acc.at[pl.ds(NS * SN, SREM)])

    plsc.subcore_barrier()
    pltpu.sync_copy(dst_hbm.at[wid], didx)

    def body(j, carry):
        base = wid * EPW + j * C
        pltpu.sync_copy(uef_hbm.at[pl.ds(base, C)], rows)
        pltpu.sync_copy(rows, acc.at[didx.at[j]], add=True)
        return carry

    lax.fori_loop(0, NCHUNK, body, 0)
    plsc.subcore_barrier()
    pltpu.sync_copy(acc.at[pl.ds(sid * SN, SN)],
                    out_hbm.at[cid, pl.ds(sid * SN, SN)])

    @pl.when(sid == NS - 1)
    def _out_rem():
        pltpu.sync_copy(acc.at[pl.ds(NS * SN, SREM)],
                        out_hbm.at[cid, pl.ds(NS * SN, SREM)])


def kernel(nf, ef, edge_index, W_node_enc, b_node_enc, W_edge_enc, b_edge_enc,
           We, be, Wn, bn):
    src3 = edge_index[0].reshape(NW, NCHUNK, C)
    dst3 = edge_index[1].reshape(NW, NCHUNK, C)
    zeros_nd = jnp.zeros((N, D), jnp.float32)

    unf, P, Q = _node_enc(nf, W_node_enc, b_node_enc.reshape(1, D),
                          We[0, D:2 * D], be[0].reshape(1, D), We[0, 2 * D:])
    uef = _edge_enc(ef, W_edge_enc, b_edge_enc.reshape(1, D))

    for l in range(N_LAYER):
        gs, gd = _sc_gather(P, Q, src3, dst3)
        uef = _edge_upd(uef, gs, gd, We[l, :D])
        partials = _sc_scatter(uef, dst3, zeros_nd)
        if l + 1 < N_LAYER:
            unf, P, Q = _node_upd_proj(
                unf, partials[0], partials[1],
                Wn[l, :D], Wn[l, D:], bn[l].reshape(1, D),
                We[l + 1, D:2 * D], be[l + 1].reshape(1, D), We[l + 1, 2 * D:])
        else:
            unf = _node_upd(unf, partials[0], partials[1],
                            Wn[l, :D], Wn[l, D:], bn[l].reshape(1, D))
    return unf, uef


# speedup vs baseline: 3.2802x; 1.1275x over previous
import functools

import jax
import jax.numpy as jnp
from jax import lax
from jax.experimental import pallas as pl
from jax.experimental.pallas import tpu as pltpu
from jax.experimental.pallas import tpu_sc as plsc

N = 10000
E = 320000
NODE_DIM = 128
EDGE_DIM = 16
D = 128
N_LAYER = 3

NC = 2
NS = 16
NW = NC * NS
EPW = E // NW
C = 80
NCHUNK = EPW // C
SN = 624
SREM = N - NS * SN

BN = 2000
BE = 2000

_mesh = plsc.VectorSubcoreMesh(
    core_axis_name="c", subcore_axis_name="s", num_cores=NC, num_subcores=NS)


def _dot(a, b):
    return jnp.dot(a, b, preferred_element_type=jnp.float32)


def _edge_enc_body(x_ref, w_ref, b_ref, o_ref):
    o_ref[...] = _dot(x_ref[...], w_ref[...]) + b_ref[...]


_edge_enc = pl.pallas_call(
    _edge_enc_body,
    grid=(E // BE,),
    in_specs=[
        pl.BlockSpec((BE, EDGE_DIM), lambda i: (i, 0)),
        pl.BlockSpec((EDGE_DIM, D), lambda i: (0, 0)),
        pl.BlockSpec((1, D), lambda i: (0, 0)),
    ],
    out_specs=pl.BlockSpec((BE, D), lambda i: (i, 0)),
    out_shape=jax.ShapeDtypeStruct((E, D), jnp.float32),
)


def _node_enc_body(x_ref, w_ref, b_ref, ws_ref, bs_ref, wd_ref,
                   u_ref, p_ref, q_ref):
    u = _dot(x_ref[...], w_ref[...]) + b_ref[...]
    u_ref[...] = u
    p_ref[...] = _dot(u, ws_ref[...]) + bs_ref[...]
    q_ref[...] = _dot(u, wd_ref[...])


_node_enc = pl.pallas_call(
    _node_enc_body,
    grid=(N // BN,),
    in_specs=[
        pl.BlockSpec((BN, NODE_DIM), lambda i: (i, 0)),
        pl.BlockSpec((NODE_DIM, D), lambda i: (0, 0)),
        pl.BlockSpec((1, D), lambda i: (0, 0)),
        pl.BlockSpec((D, D), lambda i: (0, 0)),
        pl.BlockSpec((1, D), lambda i: (0, 0)),
        pl.BlockSpec((D, D), lambda i: (0, 0)),
    ],
    out_specs=[pl.BlockSpec((BN, D), lambda i: (i, 0))] * 3,
    out_shape=[jax.ShapeDtypeStruct((N, D), jnp.float32)] * 3,
)


def _edge_upd_body(u_ref, gs_ref, gd_ref, w_ref, o_ref):
    u = u_ref[...]
    pre = _dot(u, w_ref[...]) + gs_ref[...] + gd_ref[...]
    o_ref[...] = u + jnp.maximum(pre, 0.0)


_edge_upd = pl.pallas_call(
    _edge_upd_body,
    grid=(E // BE,),
    in_specs=[
        pl.BlockSpec((BE, D), lambda i: (i, 0)),
        pl.BlockSpec((BE, D), lambda i: (i, 0)),
        pl.BlockSpec((BE, D), lambda i: (i, 0)),
        pl.BlockSpec((D, D), lambda i: (0, 0)),
    ],
    out_specs=pl.BlockSpec((BE, D), lambda i: (i, 0)),
    out_shape=jax.ShapeDtypeStruct((E, D), jnp.float32),
)


def _node_upd_proj_body(u_ref, a0_ref, a1_ref, w1_ref, w2_ref, b_ref,
                        ws_ref, bs_ref, wd_ref, uo_ref, p_ref, q_ref):
    u = u_ref[...]
    agg = a0_ref[...] + a1_ref[...]
    h = _dot(u, w1_ref[...]) + _dot(agg, w2_ref[...]) + b_ref[...]
    un = u + jnp.maximum(h, 0.0)
    uo_ref[...] = un
    p_ref[...] = _dot(un, ws_ref[...]) + bs_ref[...]
    q_ref[...] = _dot(un, wd_ref[...])


_node_upd_proj = pl.pallas_call(
    _node_upd_proj_body,
    grid=(N // BN,),
    in_specs=[
        pl.BlockSpec((BN, D), lambda i: (i, 0)),
        pl.BlockSpec((BN, D), lambda i: (i, 0)),
        pl.BlockSpec((BN, D), lambda i: (i, 0)),
        pl.BlockSpec((D, D), lambda i: (0, 0)),
        pl.BlockSpec((D, D), lambda i: (0, 0)),
        pl.BlockSpec((1, D), lambda i: (0, 0)),
        pl.BlockSpec((D, D), lambda i: (0, 0)),
        pl.BlockSpec((1, D), lambda i: (0, 0)),
        pl.BlockSpec((D, D), lambda i: (0, 0)),
    ],
    out_specs=[pl.BlockSpec((BN, D), lambda i: (i, 0))] * 3,
    out_shape=[jax.ShapeDtypeStruct((N, D), jnp.float32)] * 3,
)


def _node_upd_body(u_ref, a0_ref, a1_ref, w1_ref, w2_ref, b_ref, uo_ref):
    u = u_ref[...]
    agg = a0_ref[...] + a1_ref[...]
    h = _dot(u, w1_ref[...]) + _dot(agg, w2_ref[...]) + b_ref[...]
    uo_ref[...] = u + jnp.maximum(h, 0.0)


_node_upd = pl.pallas_call(
    _node_upd_body,
    grid=(N // BN,),
    in_specs=[
        pl.BlockSpec((BN, D), lambda i: (i, 0)),
        pl.BlockSpec((BN, D), lambda i: (i, 0)),
        pl.BlockSpec((BN, D), lambda i: (i, 0)),
        pl.BlockSpec((D, D), lambda i: (0, 0)),
        pl.BlockSpec((D, D), lambda i: (0, 0)),
        pl.BlockSpec((1, D), lambda i: (0, 0)),
    ],
    out_specs=pl.BlockSpec((BN, D), lambda i: (i, 0)),
    out_shape=jax.ShapeDtypeStruct((N, D), jnp.float32),
)


@functools.partial(
    pl.kernel,
    out_type=[jax.ShapeDtypeStruct((E, D), jnp.float32),
              jax.ShapeDtypeStruct((E, D), jnp.float32)],
    mesh=_mesh,
    scratch_types=[
        pltpu.VMEM((NCHUNK, C), jnp.int32),
        pltpu.VMEM((NCHUNK, C), jnp.int32),
        pltpu.VMEM((C, D), jnp.float32),
        pltpu.VMEM((C, D), jnp.float32),
        pltpu.VMEM((C, D), jnp.float32),
        pltpu.VMEM((C, D), jnp.float32),
        pltpu.SemaphoreType.DMA,
        pltpu.SemaphoreType.DMA,
        pltpu.SemaphoreType.DMA,
    ],
)
def _sc_gather(p_hbm, q_hbm, src_hbm, dst_hbm, gs_hbm, gd_hbm,
               sidx, didx, prow0, qrow0, prow1, qrow1, sem_p, sem_q, sem_w):
    wid = lax.axis_index("s") * NC + lax.axis_index("c")
    pltpu.sync_copy(src_hbm.at[wid], sidx)
    pltpu.sync_copy(dst_hbm.at[wid], didx)
    ebase = wid * EPW

    def start_gather(j, pbuf, qbuf):
        return (pltpu.async_copy(p_hbm.at[sidx.at[j]], pbuf, sem_p),
                pltpu.async_copy(q_hbm.at[didx.at[j]], qbuf, sem_q))

    def start_write(j, pbuf, qbuf):
        return (pltpu.async_copy(pbuf, gs_hbm.at[pl.ds(ebase + j * C, C)], sem_w),
                pltpu.async_copy(qbuf, gd_hbm.at[pl.ds(ebase + j * C, C)], sem_w))

    cp, cq = start_gather(0, prow0, qrow0)
    cp.wait()
    cq.wait()

    def body(i, carry):
        jA = 2 * i + 1
        cp, cq = start_gather(jA, prow1, qrow1)
        wp, wq = start_write(jA - 1, prow0, qrow0)
        cp.wait(); cq.wait(); wp.wait(); wq.wait()
        cp, cq = start_gather(jA + 1, prow0, qrow0)
        wp, wq = start_write(jA, prow1, qrow1)
        cp.wait(); cq.wait(); wp.wait(); wq.wait()
        return carry

    lax.fori_loop(0, (NCHUNK - 1) // 2, body, 0)
    wp, wq = start_write(NCHUNK - 1, prow0, qrow0)
    wp.wait()
    wq.wait()


@functools.partial(
    pl.kernel,
    out_type=jax.ShapeDtypeStruct((NC, N, D), jnp.float32),
    mesh=_mesh,
    scratch_types=[
        pltpu.VMEM((NCHUNK, C), jnp.int32),
        pltpu.VMEM((C, D), jnp.float32),
        pltpu.VMEM((C, D), jnp.float32),
        pltpu.VMEM_SHARED((N, D), jnp.float32),
        pltpu.SemaphoreType.DMA,
        pltpu.SemaphoreType.DMA,
    ],
)
def _sc_scatter(uef_hbm, dst_hbm, zeros_hbm, out_hbm,
                didx, rows0, rows1, acc, sem_l, sem_s):
    cid = lax.axis_index("c")
    sid = lax.axis_index("s")
    wid = sid * NC + cid
    pltpu.sync_copy(zeros_hbm.at[pl.ds(sid * SN, SN)],
                    acc.at[pl.ds(sid * SN, SN)])

    @pl.when(sid == NS - 1)
    def _zero_rem():
        pltpu.sync_copy(zeros_hbm.at[pl.ds(NS * SN, SREM)],
                        acc.at[pl.ds(NS * SN, SREM)])

    plsc.subcore_barrier()
    pltpu.sync_copy(dst_hbm.at[wid], didx)
    ebase = wid * EPW

    def start_load(j, buf):
        return pltpu.async_copy(uef_hbm.at[pl.ds(ebase + j * C, C)], buf, sem_l)

    def start_scat(j, buf):
        return pltpu.async_copy(buf, acc.at[didx.at[j]], sem_s, add=True)

    start_load(0, rows0).wait()

    def body(i, carry):
        jA = 2 * i + 1
        lA = start_load(jA, rows1)
        sP = start_scat(jA - 1, rows0)
        lA.wait(); sP.wait()
        lB = start_load(jA + 1, rows0)
        sA = start_scat(jA, rows1)
        lB.wait(); sA.wait()
        return carry

    lax.fori_loop(0, (NCHUNK - 1) // 2, body, 0)
    start_scat(NCHUNK - 1, rows0).wait()
    plsc.subcore_barrier()
    pltpu.sync_copy(acc.at[pl.ds(sid * SN, SN)],
                    out_hbm.at[cid, pl.ds(sid * SN, SN)])

    @pl.when(sid == NS - 1)
    def _out_rem():
        pltpu.sync_copy(acc.at[pl.ds(NS * SN, SREM)],
                        out_hbm.at[cid, pl.ds(NS * SN, SREM)])


def kernel(nf, ef, edge_index, W_node_enc, b_node_enc, W_edge_enc, b_edge_enc,
           We, be, Wn, bn):
    src3 = edge_index[0].reshape(NW, NCHUNK, C)
    dst3 = edge_index[1].reshape(NW, NCHUNK, C)
    zeros_nd = jnp.zeros((N, D), jnp.float32)

    unf, P, Q = _node_enc(nf, W_node_enc, b_node_enc.reshape(1, D),
                          We[0, D:2 * D], be[0].reshape(1, D), We[0, 2 * D:])
    uef = _edge_enc(ef, W_edge_enc, b_edge_enc.reshape(1, D))

    for l in range(N_LAYER):
        gs, gd = _sc_gather(P, Q, src3, dst3)
        uef = _edge_upd(uef, gs, gd, We[l, :D])
        partials = _sc_scatter(uef, dst3, zeros_nd)
        if l + 1 < N_LAYER:
            unf, P, Q = _node_upd_proj(
                unf, partials[0], partials[1],
                Wn[l, :D], Wn[l, D:], bn[l].reshape(1, D),
                We[l + 1, D:2 * D], be[l + 1].reshape(1, D), We[l + 1, 2 * D:])
        else:
            unf = _node_upd(unf, partials[0], partials[1],
                            Wn[l, :D], Wn[l, D:], bn[l].reshape(1, D))
    return unf, uef
